# Initial kernel scaffold; baseline (speedup 1.0000x reference)
import jax
import jax.numpy as jnp
from jax.experimental import pallas as pl


def kernel(obs, W1, b1, W2, b2, Wl1, Wr1, att1, bias1, Wl2, Wr2, att2, bias2, Wo, bo):
    raise NotImplementedError("write your pallas kernel here")


# speedup vs baseline: 2.4328x; 2.4328x over previous
import jax
import jax.numpy as jnp
from jax.experimental import pallas as pl

N = 32
F_IN = 16
D = 128
H = 2
GB = 8
RADIUS = 0.3

_INTERPRET = False


def _gat_block(x2d, adj, wl, wr, att, bias, gb):
    xl = jnp.dot(x2d, wl, preferred_element_type=jnp.float32)
    xr = jnp.dot(x2d, wr, preferred_element_type=jnp.float32)
    outs = []
    neg = jnp.finfo(jnp.float32).min
    for h in range(H):
        xl_h = xl[:, h * D:(h + 1) * D].reshape(gb, N, D)
        xr_h = xr[:, h * D:(h + 1) * D].reshape(gb, N, D)
        s = xr_h[:, :, None, :] + xl_h[:, None, :, :]
        s = jnp.where(s > 0, s, 0.2 * s)
        att_h = att[h:h + 1, :].reshape(1, 1, 1, D)
        e = jnp.sum(s * att_h, axis=3)
        e = jnp.where(adj, e, neg)
        emax = jnp.max(e, axis=2, keepdims=True)
        ee = jnp.where(adj, jnp.exp(e - emax), 0.0)
        denom = jnp.maximum(jnp.sum(ee, axis=2, keepdims=True), 1e-16)
        alpha = ee / denom
        out_h = jax.lax.dot_general(
            alpha, xl_h, (((2,), (1,)), ((0,), (0,))),
            preferred_element_type=jnp.float32)
        outs.append(out_h)
    out = jnp.concatenate(outs, axis=2)
    return out.reshape(gb * N, H * D) + bias


def _fused_kernel(feats_ref, px_ref, py_ref, idx_ref,
                  W1_ref, b1_ref, W2_ref, b2_ref,
                  Wl1_ref, Wr1_ref, att1_ref, bias1_ref,
                  Wl2_ref, Wr2_ref, att2_ref, bias2_ref,
                  Wo_ref, bo_ref, out_ref):
    gb = GB
    feats = feats_ref[...]
    px = px_ref[...]
    py = py_ref[...]
    idx = idx_ref[...]

    dx = px[:, :, None] - px[:, None, :]
    dy = py[:, :, None] - py[:, None, :]
    d2 = dx * dx + dy * dy
    ii = jax.lax.broadcasted_iota(jnp.int32, (gb, N, N), 1)
    jj = jax.lax.broadcasted_iota(jnp.int32, (gb, N, N), 2)
    adj = (d2 <= RADIUS * RADIUS) & (ii != jj)

    kk = jax.lax.broadcasted_iota(jnp.int32, (gb, N), 1)
    oh = jnp.where(kk == idx, 1.0, 0.0)

    h1 = jnp.maximum(
        jnp.dot(feats, W1_ref[...], preferred_element_type=jnp.float32)
        + b1_ref[...], 0.0)
    x = jnp.maximum(
        jnp.dot(h1, W2_ref[...], preferred_element_type=jnp.float32)
        + b2_ref[...], 0.0)
    x1 = jnp.sum(oh[:, :, None] * x.reshape(gb, N, D), axis=1)

    g1 = jnp.maximum(
        _gat_block(x, adj, Wl1_ref[...], Wr1_ref[...],
                   att1_ref[...], bias1_ref[...], gb), 0.0)
    x2 = jnp.sum(oh[:, :, None] * g1.reshape(gb, N, H * D), axis=1)

    g2 = jnp.maximum(
        _gat_block(g1, adj, Wl2_ref[...], Wr2_ref[...],
                   att2_ref[...], bias2_ref[...], gb), 0.0)
    x3 = jnp.sum(oh[:, :, None] * g2.reshape(gb, N, H * D), axis=1)

    xcat = jnp.concatenate([x1, x2, x3], axis=1)
    out_ref[...] = (jnp.dot(xcat, Wo_ref[...],
                            preferred_element_type=jnp.float32)
                    + bo_ref[...])


def kernel(obs, W1, b1, W2, b2, Wl1, Wr1, att1, bias1,
           Wl2, Wr2, att2, bias2, Wo, bo):
    bs = obs.shape[0]
    per = 2 + F_IN
    node = obs[:, :-1].reshape(bs, N, per)
    px = node[:, :, 0]
    py = node[:, :, 1]
    feats = node[:, :, 2:].reshape(bs * N, F_IN)
    idx = jnp.clip(obs[:, -1], 0, N - 1).astype(jnp.int32).reshape(bs, 1)

    grid = bs // GB
    const = lambda i: (0, 0)
    row = lambda i: (i, 0)
    out = pl.pallas_call(
        _fused_kernel,
        grid=(grid,),
        in_specs=[
            pl.BlockSpec((GB * N, F_IN), row),
            pl.BlockSpec((GB, N), row),
            pl.BlockSpec((GB, N), row),
            pl.BlockSpec((GB, 1), row),
            pl.BlockSpec((F_IN, D), const),
            pl.BlockSpec((1, D), const),
            pl.BlockSpec((D, D), const),
            pl.BlockSpec((1, D), const),
            pl.BlockSpec((D, H * D), const),
            pl.BlockSpec((D, H * D), const),
            pl.BlockSpec((H, D), const),
            pl.BlockSpec((1, H * D), const),
            pl.BlockSpec((H * D, H * D), const),
            pl.BlockSpec((H * D, H * D), const),
            pl.BlockSpec((H, D), const),
            pl.BlockSpec((1, H * D), const),
            pl.BlockSpec((D + 2 * H * D, 10), const),
            pl.BlockSpec((1, 10), const),
        ],
        out_specs=pl.BlockSpec((GB, 10), row),
        out_shape=jax.ShapeDtypeStruct((bs, 10), jnp.float32),
        interpret=_INTERPRET,
    )(feats, px, py, idx,
      W1, b1.reshape(1, D), W2, b2.reshape(1, D),
      Wl1, Wr1, att1, bias1.reshape(1, H * D),
      Wl2, Wr2, att2, bias2.reshape(1, H * D),
      Wo, bo.reshape(1, 10))
    return out


# speedup vs baseline: 2.7434x; 1.1276x over previous
import jax
import jax.numpy as jnp
from jax.experimental import pallas as pl

N = 32
F_IN = 16
D = 128
H = 2
GB = 16
RADIUS = 0.3

_INTERPRET = False


def _gat_block(x2d, adj, wl, wr, att, bias, gb):
    xl = jnp.dot(x2d, wl, preferred_element_type=jnp.float32)
    xr = jnp.dot(x2d, wr, preferred_element_type=jnp.float32)
    outs = []
    neg = jnp.finfo(jnp.float32).min
    for h in range(H):
        xl_h = xl[:, h * D:(h + 1) * D].reshape(gb, N, D)
        xr_h = xr[:, h * D:(h + 1) * D].reshape(gb, N, D)
        s = xr_h[:, :, None, :] + xl_h[:, None, :, :]
        s = jnp.where(s > 0, s, 0.2 * s)
        att_h = att[h:h + 1, :].reshape(1, 1, 1, D)
        e = jnp.sum(s * att_h, axis=3)
        e = jnp.where(adj, e, neg)
        emax = jnp.max(e, axis=2, keepdims=True)
        ee = jnp.where(adj, jnp.exp(e - emax), 0.0)
        denom = jnp.maximum(jnp.sum(ee, axis=2, keepdims=True), 1e-16)
        alpha = ee / denom
        out_h = jax.lax.dot_general(
            alpha, xl_h, (((2,), (1,)), ((0,), (0,))),
            preferred_element_type=jnp.float32)
        outs.append(out_h)
    out = jnp.concatenate(outs, axis=2)
    return out.reshape(gb * N, H * D) + bias


def _fused_kernel(feats_ref, px_ref, py_ref, idx_ref,
                  W1_ref, b1_ref, W2_ref, b2_ref,
                  Wl1_ref, Wr1_ref, att1_ref, bias1_ref,
                  Wl2_ref, Wr2_ref, att2_ref, bias2_ref,
                  Wo_ref, bo_ref, out_ref):
    gb = GB
    feats = feats_ref[...]
    px = px_ref[...]
    py = py_ref[...]
    idx = idx_ref[...]

    dx = px[:, :, None] - px[:, None, :]
    dy = py[:, :, None] - py[:, None, :]
    d2 = dx * dx + dy * dy
    ii = jax.lax.broadcasted_iota(jnp.int32, (gb, N, N), 1)
    jj = jax.lax.broadcasted_iota(jnp.int32, (gb, N, N), 2)
    adj = (d2 <= RADIUS * RADIUS) & (ii != jj)

    kk = jax.lax.broadcasted_iota(jnp.int32, (gb, N), 1)
    oh = jnp.where(kk == idx, 1.0, 0.0)

    h1 = jnp.maximum(
        jnp.dot(feats, W1_ref[...], preferred_element_type=jnp.float32)
        + b1_ref[...], 0.0)
    x = jnp.maximum(
        jnp.dot(h1, W2_ref[...], preferred_element_type=jnp.float32)
        + b2_ref[...], 0.0)
    x1 = jnp.sum(oh[:, :, None] * x.reshape(gb, N, D), axis=1)

    g1 = jnp.maximum(
        _gat_block(x, adj, Wl1_ref[...], Wr1_ref[...],
                   att1_ref[...], bias1_ref[...], gb), 0.0)
    x2 = jnp.sum(oh[:, :, None] * g1.reshape(gb, N, H * D), axis=1)

    g2 = jnp.maximum(
        _gat_block(g1, adj, Wl2_ref[...], Wr2_ref[...],
                   att2_ref[...], bias2_ref[...], gb), 0.0)
    x3 = jnp.sum(oh[:, :, None] * g2.reshape(gb, N, H * D), axis=1)

    xcat = jnp.concatenate([x1, x2, x3], axis=1)
    out_ref[...] = (jnp.dot(xcat, Wo_ref[...],
                            preferred_element_type=jnp.float32)
                    + bo_ref[...])


def kernel(obs, W1, b1, W2, b2, Wl1, Wr1, att1, bias1,
           Wl2, Wr2, att2, bias2, Wo, bo):
    bs = obs.shape[0]
    per = 2 + F_IN
    node = obs[:, :-1].reshape(bs, N, per)
    px = node[:, :, 0]
    py = node[:, :, 1]
    feats = node[:, :, 2:].reshape(bs * N, F_IN)
    idx = jnp.clip(obs[:, -1], 0, N - 1).astype(jnp.int32).reshape(bs, 1)

    grid = bs // GB
    const = lambda i: (0, 0)
    row = lambda i: (i, 0)
    out = pl.pallas_call(
        _fused_kernel,
        grid=(grid,),
        in_specs=[
            pl.BlockSpec((GB * N, F_IN), row),
            pl.BlockSpec((GB, N), row),
            pl.BlockSpec((GB, N), row),
            pl.BlockSpec((GB, 1), row),
            pl.BlockSpec((F_IN, D), const),
            pl.BlockSpec((1, D), const),
            pl.BlockSpec((D, D), const),
            pl.BlockSpec((1, D), const),
            pl.BlockSpec((D, H * D), const),
            pl.BlockSpec((D, H * D), const),
            pl.BlockSpec((H, D), const),
            pl.BlockSpec((1, H * D), const),
            pl.BlockSpec((H * D, H * D), const),
            pl.BlockSpec((H * D, H * D), const),
            pl.BlockSpec((H, D), const),
            pl.BlockSpec((1, H * D), const),
            pl.BlockSpec((D + 2 * H * D, 10), const),
            pl.BlockSpec((1, 10), const),
        ],
        out_specs=pl.BlockSpec((GB, 10), row),
        out_shape=jax.ShapeDtypeStruct((bs, 10), jnp.float32),
        interpret=_INTERPRET,
    )(feats, px, py, idx,
      W1, b1.reshape(1, D), W2, b2.reshape(1, D),
      Wl1, Wr1, att1, bias1.reshape(1, H * D),
      Wl2, Wr2, att2, bias2.reshape(1, H * D),
      Wo, bo.reshape(1, 10))
    return out


# speedup vs baseline: 2.8722x; 1.0470x over previous
import jax
import jax.numpy as jnp
from jax.experimental import pallas as pl

N = 32
F_IN = 16
D = 128
H = 2
GB = 32
RADIUS = 0.3

_INTERPRET = False


def _gat_block(x2d, adj, wl, wr, att, bias, gb):
    xl = jnp.dot(x2d, wl, preferred_element_type=jnp.float32)
    xr = jnp.dot(x2d, wr, preferred_element_type=jnp.float32)
    outs = []
    neg = jnp.finfo(jnp.float32).min
    for h in range(H):
        xl_h = xl[:, h * D:(h + 1) * D].reshape(gb, N, D)
        xr_h = xr[:, h * D:(h + 1) * D].reshape(gb, N, D)
        s = xr_h[:, :, None, :] + xl_h[:, None, :, :]
        s = jnp.where(s > 0, s, 0.2 * s)
        att_h = att[h:h + 1, :].reshape(1, 1, 1, D)
        e = jnp.sum(s * att_h, axis=3)
        e = jnp.where(adj, e, neg)
        emax = jnp.max(e, axis=2, keepdims=True)
        ee = jnp.where(adj, jnp.exp(e - emax), 0.0)
        denom = jnp.maximum(jnp.sum(ee, axis=2, keepdims=True), 1e-16)
        alpha = ee / denom
        out_h = jax.lax.dot_general(
            alpha, xl_h, (((2,), (1,)), ((0,), (0,))),
            preferred_element_type=jnp.float32)
        outs.append(out_h)
    out = jnp.concatenate(outs, axis=2)
    return out.reshape(gb * N, H * D) + bias


def _fused_kernel(feats_ref, px_ref, py_ref, idx_ref,
                  W1_ref, b1_ref, W2_ref, b2_ref,
                  Wl1_ref, Wr1_ref, att1_ref, bias1_ref,
                  Wl2_ref, Wr2_ref, att2_ref, bias2_ref,
                  Wo_ref, bo_ref, out_ref):
    gb = GB
    feats = feats_ref[...]
    px = px_ref[...]
    py = py_ref[...]
    idx = idx_ref[...]

    dx = px[:, :, None] - px[:, None, :]
    dy = py[:, :, None] - py[:, None, :]
    d2 = dx * dx + dy * dy
    ii = jax.lax.broadcasted_iota(jnp.int32, (gb, N, N), 1)
    jj = jax.lax.broadcasted_iota(jnp.int32, (gb, N, N), 2)
    adj = (d2 <= RADIUS * RADIUS) & (ii != jj)

    kk = jax.lax.broadcasted_iota(jnp.int32, (gb, N), 1)
    oh = jnp.where(kk == idx, 1.0, 0.0)

    h1 = jnp.maximum(
        jnp.dot(feats, W1_ref[...], preferred_element_type=jnp.float32)
        + b1_ref[...], 0.0)
    x = jnp.maximum(
        jnp.dot(h1, W2_ref[...], preferred_element_type=jnp.float32)
        + b2_ref[...], 0.0)
    x1 = jnp.sum(oh[:, :, None] * x.reshape(gb, N, D), axis=1)

    g1 = jnp.maximum(
        _gat_block(x, adj, Wl1_ref[...], Wr1_ref[...],
                   att1_ref[...], bias1_ref[...], gb), 0.0)
    x2 = jnp.sum(oh[:, :, None] * g1.reshape(gb, N, H * D), axis=1)

    g2 = jnp.maximum(
        _gat_block(g1, adj, Wl2_ref[...], Wr2_ref[...],
                   att2_ref[...], bias2_ref[...], gb), 0.0)
    x3 = jnp.sum(oh[:, :, None] * g2.reshape(gb, N, H * D), axis=1)

    xcat = jnp.concatenate([x1, x2, x3], axis=1)
    out_ref[...] = (jnp.dot(xcat, Wo_ref[...],
                            preferred_element_type=jnp.float32)
                    + bo_ref[...])


def kernel(obs, W1, b1, W2, b2, Wl1, Wr1, att1, bias1,
           Wl2, Wr2, att2, bias2, Wo, bo):
    bs = obs.shape[0]
    per = 2 + F_IN
    node = obs[:, :-1].reshape(bs, N, per)
    px = node[:, :, 0]
    py = node[:, :, 1]
    feats = node[:, :, 2:].reshape(bs * N, F_IN)
    idx = jnp.clip(obs[:, -1], 0, N - 1).astype(jnp.int32).reshape(bs, 1)

    grid = bs // GB
    const = lambda i: (0, 0)
    row = lambda i: (i, 0)
    out = pl.pallas_call(
        _fused_kernel,
        grid=(grid,),
        in_specs=[
            pl.BlockSpec((GB * N, F_IN), row),
            pl.BlockSpec((GB, N), row),
            pl.BlockSpec((GB, N), row),
            pl.BlockSpec((GB, 1), row),
            pl.BlockSpec((F_IN, D), const),
            pl.BlockSpec((1, D), const),
            pl.BlockSpec((D, D), const),
            pl.BlockSpec((1, D), const),
            pl.BlockSpec((D, H * D), const),
            pl.BlockSpec((D, H * D), const),
            pl.BlockSpec((H, D), const),
            pl.BlockSpec((1, H * D), const),
            pl.BlockSpec((H * D, H * D), const),
            pl.BlockSpec((H * D, H * D), const),
            pl.BlockSpec((H, D), const),
            pl.BlockSpec((1, H * D), const),
            pl.BlockSpec((D + 2 * H * D, 10), const),
            pl.BlockSpec((1, 10), const),
        ],
        out_specs=pl.BlockSpec((GB, 10), row),
        out_shape=jax.ShapeDtypeStruct((bs, 10), jnp.float32),
        interpret=_INTERPRET,
    )(feats, px, py, idx,
      W1, b1.reshape(1, D), W2, b2.reshape(1, D),
      Wl1, Wr1, att1, bias1.reshape(1, H * D),
      Wl2, Wr2, att2, bias2.reshape(1, H * D),
      Wo, bo.reshape(1, 10))
    return out
